# baseline (speedup 1.0000x reference)
import functools

import jax
import jax.numpy as jnp
from jax.experimental import pallas as pl
from jax.experimental.pallas import tpu as pltpu


def _do_half(a_ref, h_ref, wt_ref, o_ref, row0, half, out_off):
    a = a_ref[...]
    s = jax.lax.dot_general(
        a, h_ref[...], (((1,), (0,)), ((), ())),
        precision=jax.lax.Precision.DEFAULT,
        preferred_element_type=jnp.float32,
    )
    deg = jnp.sum(a, axis=1, keepdims=True)
    hr = h_ref[pl.ds(row0, half), :]
    agg = (s + hr) / (deg + 1.0)
    z = jnp.dot(agg, wt_ref[...], preferred_element_type=jnp.float32)
    o_ref[pl.ds(out_off, half), :] = jnp.where(z >= 0.0, z, 0.01 * z)


def _sage_kernel2(a0_ref, a1_ref, h_ref, wt_ref, o_ref, *, half):
    i = pl.program_id(0)
    base = i * 2 * half
    _do_half(a0_ref, h_ref, wt_ref, o_ref, base, half, 0)
    _do_half(a1_ref, h_ref, wt_ref, o_ref, base + half, half, half)


def _sage_kernel1(a_ref, h_ref, wt_ref, o_ref, *, half):
    i = pl.program_id(0)
    _do_half(a_ref, h_ref, wt_ref, o_ref, i * half, half, 0)


def kernel(A, h, weight):
    n, d = h.shape
    wt = weight.T
    half = 200

    if n % (2 * half) == 0:
        out = pl.pallas_call(
            functools.partial(_sage_kernel2, half=half),
            grid=(n // (2 * half),),
            in_specs=[
                pl.BlockSpec((half, n), lambda i: (2 * i, 0)),
                pl.BlockSpec((half, n), lambda i: (2 * i + 1, 0)),
                pl.BlockSpec((n, d), lambda i: (0, 0)),
                pl.BlockSpec((d, d), lambda i: (0, 0)),
            ],
            out_specs=pl.BlockSpec((2 * half, d), lambda i: (i, 0)),
            out_shape=jax.ShapeDtypeStruct((n, d), jnp.float32),
            compiler_params=pltpu.CompilerParams(
                dimension_semantics=("arbitrary",),
            ),
        )(A, A, h, wt)
        return out

    blk = n if n % 8 else min(n, 400)
    out = pl.pallas_call(
        functools.partial(_sage_kernel1, half=blk),
        grid=(n // blk,),
        in_specs=[
            pl.BlockSpec((blk, n), lambda i: (i, 0)),
            pl.BlockSpec((n, d), lambda i: (0, 0)),
            pl.BlockSpec((d, d), lambda i: (0, 0)),
        ],
        out_specs=pl.BlockSpec((blk, d), lambda i: (i, 0)),
        out_shape=jax.ShapeDtypeStruct((n, d), jnp.float32),
    )(A, h, wt)
    return out


# speedup vs baseline: 1.0967x; 1.0967x over previous
import functools

import jax
import jax.numpy as jnp
from jax.experimental import pallas as pl
from jax.experimental.pallas import tpu as pltpu


def _sage_kernel(a_ref, h_ref, wt_ref, o_ref, *, row_blk):
    i = pl.program_id(0)
    a = a_ref[...]
    s = jax.lax.dot_general(
        a, h_ref[...], (((1,), (0,)), ((), ())),
        precision=jax.lax.Precision.DEFAULT,
        preferred_element_type=jnp.float32,
    )
    deg = jnp.sum(a, axis=1, keepdims=True)
    hr = h_ref[pl.ds(i * row_blk, row_blk), :]
    agg = (s + hr) / (deg + 1.0)
    z = jnp.dot(agg, wt_ref[...], preferred_element_type=jnp.float32)
    o_ref[...] = jnp.where(z >= 0.0, z, 0.01 * z)


def _pick_block(n, target):
    if n % target == 0:
        return target
    return n


def kernel(A, h, weight):
    n, d = h.shape
    row_blk = _pick_block(n, 400)
    wt = weight.T

    out = pl.pallas_call(
        functools.partial(_sage_kernel, row_blk=row_blk),
        grid=(n // row_blk,),
        in_specs=[
            pl.BlockSpec((row_blk, n), lambda i: (i, 0)),
            pl.BlockSpec((n, d), lambda i: (0, 0)),
            pl.BlockSpec((d, d), lambda i: (0, 0)),
        ],
        out_specs=pl.BlockSpec((row_blk, d), lambda i: (i, 0)),
        out_shape=jax.ShapeDtypeStruct((n, d), jnp.float32),
        compiler_params=pltpu.CompilerParams(
            dimension_semantics=("parallel",),
        ),
    )(A, h, wt)
    return out
